# Initial kernel scaffold; baseline (speedup 1.0000x reference)
import functools

import jax
import jax.numpy as jnp
from jax import lax
from jax.experimental import pallas as pl
from jax.experimental.pallas import tpu as pltpu
from jax.experimental.pallas import tpu_sc as plsc

NC = 2
NS = 16
NW = NC * NS
L = 16
D = 128
CG = 128


def _mesh():
    return plsc.VectorSubcoreMesh(
        core_axis_name="c", subcore_axis_name="s", num_cores=NC, num_subcores=NS)


def _make_deg_kernel(nchunk, npad):
    @functools.partial(
        pl.kernel, mesh=_mesh(),
        out_type=jax.ShapeDtypeStruct((NW, npad), jnp.float32),
        scratch_types=[
            pltpu.VMEM((nchunk, CG), jnp.int32),
            pltpu.VMEM((nchunk, CG), jnp.float32),
            pltpu.VMEM((npad,), jnp.float32),
        ])
    def deg_kernel(col_hbm, ew_hbm, degp_hbm, col_v, ew_v, deg_v):
        w = lax.axis_index("s") * NC + lax.axis_index("c")
        pltpu.sync_copy(col_hbm.at[w], col_v)
        pltpu.sync_copy(ew_hbm.at[w], ew_v)

        def zero_body(i, carry):
            deg_v[pl.ds(i * L, L)] = jnp.zeros((L,), jnp.float32)
            return carry
        lax.fori_loop(0, npad // L, zero_body, 0)

        def chunk_body(c, carry):
            def sub(j, carry2):
                c16 = col_v[c, pl.ds(j * L, L)]
                e16 = ew_v[c, pl.ds(j * L, L)]
                plsc.addupdate_scatter(deg_v, [c16], e16)
                return carry2
            return lax.fori_loop(0, CG // L, sub, carry)
        lax.fori_loop(0, nchunk, chunk_body, 0)
        pltpu.sync_copy(deg_v, degp_hbm.at[w])

    return deg_kernel


def _make_dinv_kernel(npad):
    npt = npad // NW

    @functools.partial(
        pl.kernel, mesh=_mesh(),
        out_type=jax.ShapeDtypeStruct((npad,), jnp.float32),
        scratch_types=[
            pltpu.VMEM((NW, npt), jnp.float32),
            pltpu.VMEM((npt,), jnp.float32),
        ])
    def dinv_kernel(degp_hbm, dinv_hbm, degs_v, dinv_v):
        w = lax.axis_index("s") * NC + lax.axis_index("c")
        pltpu.sync_copy(degp_hbm.at[:, pl.ds(w * npt, npt)], degs_v)

        def body(j, carry):
            acc = jnp.zeros((L,), jnp.float32)
            for r in range(NW):
                acc = acc + degs_v[r, pl.ds(j * L, L)]
            i32 = plsc.bitcast(acc, jnp.int32)
            i32 = jnp.int32(0x5F3759DF) - lax.shift_right_arithmetic(i32, 1)
            y = plsc.bitcast(i32, jnp.float32)
            for _ in range(3):
                y = y * (1.5 - 0.5 * acc * y * y)
            dinv_v[pl.ds(j * L, L)] = y
            return carry
        lax.fori_loop(0, npt // L, body, 0)
        pltpu.sync_copy(dinv_v, dinv_hbm.at[pl.ds(w * npt, npt)])

    return dinv_kernel


def _make_agg_kernel(nchunk, npad):
    spt = npad // NS

    @functools.partial(
        pl.kernel, mesh=_mesh(),
        out_type=jax.ShapeDtypeStruct((NC, npad, D), jnp.float32),
        scratch_types=[
            pltpu.VMEM((nchunk, CG), jnp.int32),
            pltpu.VMEM((nchunk, CG), jnp.int32),
            pltpu.VMEM((nchunk, CG), jnp.float32),
            pltpu.VMEM((npad,), jnp.float32),
            pltpu.VMEM((CG,), jnp.float32),
            pltpu.VMEM((CG, D), jnp.float32),
            pltpu.VMEM_SHARED((npad, D), jnp.float32),
            pltpu.SemaphoreType.DMA,
        ])
    def agg_kernel(src_hbm, row_hbm, col_hbm, ew_hbm, dinv_hbm, out_hbm,
                   row_v, col_v, ew_v, dinv_v, norm_v, rows_v, accum, sem):
        cid = lax.axis_index("c")
        sid = lax.axis_index("s")
        w = sid * NC + cid
        pltpu.sync_copy(row_hbm.at[w], row_v)
        pltpu.sync_copy(col_hbm.at[w], col_v)
        pltpu.sync_copy(ew_hbm.at[w], ew_v)
        pltpu.sync_copy(dinv_hbm, dinv_v)

        def zr(i, carry):
            for j in range(D // L):
                rows_v[i, pl.ds(j * L, L)] = jnp.zeros((L,), jnp.float32)
            return carry
        lax.fori_loop(0, CG, zr, 0)
        for b in range(spt // CG):
            pltpu.sync_copy(rows_v, accum.at[pl.ds(sid * spt + b * CG, CG)])
        plsc.subcore_barrier()

        def chunk(c, carry):
            def nb(j, carry2):
                r16 = row_v[c, pl.ds(j * L, L)]
                c16 = col_v[c, pl.ds(j * L, L)]
                e16 = ew_v[c, pl.ds(j * L, L)]
                dr = plsc.load_gather(dinv_v, [r16])
                dc = plsc.load_gather(dinv_v, [c16])
                norm_v[pl.ds(j * L, L)] = dr * e16 * dc
                return carry2
            lax.fori_loop(0, CG // L, nb, 0)
            pltpu.async_copy(src_hbm.at[row_v.at[c]], rows_v, sem).wait()

            def sc_e(e, carry2):
                nbv = plsc.load_gather(norm_v, [jnp.zeros((L,), jnp.int32) + e])
                for j in range(D // L):
                    rows_v[e, pl.ds(j * L, L)] = rows_v[e, pl.ds(j * L, L)] * nbv
                return carry2
            lax.fori_loop(0, CG, sc_e, 0)
            pltpu.sync_copy(rows_v, accum.at[col_v.at[c]], add=True)
            return carry
        lax.fori_loop(0, nchunk, chunk, 0)
        plsc.subcore_barrier()
        pltpu.sync_copy(accum.at[pl.ds(sid * spt, spt)],
                        out_hbm.at[cid, pl.ds(sid * spt, spt)])

    return agg_kernel


def _relu_mm(p_ref, w_ref, b_ref, o_ref):
    a = p_ref[0] + p_ref[1]
    o_ref[...] = jnp.maximum(
        jnp.dot(a, w_ref[...], preferred_element_type=jnp.float32) + b_ref[...],
        0.0)


def _mm2(p_ref, w2_ref, b2_ref, w3_ref, b3_ref, m_ref, v_ref):
    a = p_ref[0] + p_ref[1]
    m_ref[...] = jnp.dot(a, w2_ref[...], preferred_element_type=jnp.float32) + b2_ref[...]
    v_ref[...] = jnp.dot(a, w3_ref[...], preferred_element_type=jnp.float32) + b3_ref[...]


def kernel(x, edge_index, edge_weight, W1, b1, W2, b2, W3, b3):
    n, d_in = x.shape
    e = edge_index.shape[1]
    npad = -(-n // 512) * 512
    etot = e + n
    nchunk = -(-etot // (NW * CG))
    epad = NW * CG * nchunk

    loop = jnp.arange(n, dtype=jnp.int32)
    row = jnp.concatenate([edge_index[0], loop])
    col = jnp.concatenate([edge_index[1], loop])
    ew = jnp.concatenate([edge_weight, jnp.ones((n,), jnp.float32)])
    pad = epad - etot
    row3 = jnp.pad(row, (0, pad)).reshape(NW, nchunk, CG)
    col3 = jnp.pad(col, (0, pad)).reshape(NW, nchunk, CG)
    ew3 = jnp.pad(ew, (0, pad)).reshape(NW, nchunk, CG)

    degp = _make_deg_kernel(nchunk, npad)(col3, ew3)
    dinv = _make_dinv_kernel(npad)(degp)

    agg = _make_agg_kernel(nchunk, npad)
    p1 = agg(x, row3, col3, ew3, dinv)

    bn = 512
    h = pl.pallas_call(
        _relu_mm,
        grid=(npad // bn,),
        in_specs=[
            pl.BlockSpec((NC, bn, D), lambda i: (0, i, 0)),
            pl.BlockSpec((D, D), lambda i: (0, 0)),
            pl.BlockSpec((1, D), lambda i: (0, 0)),
        ],
        out_specs=pl.BlockSpec((bn, D), lambda i: (i, 0)),
        out_shape=jax.ShapeDtypeStruct((npad, D), jnp.float32),
    )(p1, W1, b1.reshape(1, D))

    p2 = agg(h, row3, col3, ew3, dinv)

    mean, var = pl.pallas_call(
        _mm2,
        grid=(npad // bn,),
        in_specs=[
            pl.BlockSpec((NC, bn, D), lambda i: (0, i, 0)),
            pl.BlockSpec((D, D), lambda i: (0, 0)),
            pl.BlockSpec((1, D), lambda i: (0, 0)),
            pl.BlockSpec((D, D), lambda i: (0, 0)),
            pl.BlockSpec((1, D), lambda i: (0, 0)),
        ],
        out_specs=[
            pl.BlockSpec((bn, D), lambda i: (i, 0)),
            pl.BlockSpec((bn, D), lambda i: (i, 0)),
        ],
        out_shape=[
            jax.ShapeDtypeStruct((npad, D), jnp.float32),
            jax.ShapeDtypeStruct((npad, D), jnp.float32),
        ],
    )(p2, W2, b2.reshape(1, D), W3, b3.reshape(1, D))

    return (mean[:n], var[:n])


# speedup vs baseline: 11.6527x; 11.6527x over previous
import functools

import jax
import jax.numpy as jnp
from jax import lax
from jax.experimental import pallas as pl
from jax.experimental.pallas import tpu as pltpu
from jax.experimental.pallas import tpu_sc as plsc

NC = 2
NS = 16
NW = NC * NS
L = 16
D = 128
CG = 128


def _mesh():
    return plsc.VectorSubcoreMesh(
        core_axis_name="c", subcore_axis_name="s", num_cores=NC, num_subcores=NS)


_SC_PARAMS = pltpu.CompilerParams(needs_layout_passes=False)


def _make_deg_kernel(nchunk, npad):
    @functools.partial(
        pl.kernel, mesh=_mesh(), compiler_params=_SC_PARAMS,
        out_type=jax.ShapeDtypeStruct((NW * npad,), jnp.float32),
        scratch_types=[
            pltpu.VMEM((nchunk, CG), jnp.int32),
            pltpu.VMEM((nchunk, CG), jnp.float32),
            pltpu.VMEM((npad,), jnp.float32),
        ])
    def deg_kernel(col_hbm, ew_hbm, degp_hbm, col_v, ew_v, deg_v):
        w = lax.axis_index("s") * NC + lax.axis_index("c")
        pltpu.sync_copy(col_hbm.at[w], col_v)
        pltpu.sync_copy(ew_hbm.at[w], ew_v)

        def zero_body(i, carry):
            deg_v[pl.ds(i * L, L)] = jnp.zeros((L,), jnp.float32)
            return carry
        lax.fori_loop(0, npad // L, zero_body, 0)

        def chunk_body(c, carry):
            def sub(j, carry2):
                c16 = col_v[c, pl.ds(j * L, L)]
                e16 = ew_v[c, pl.ds(j * L, L)]
                plsc.addupdate_scatter(deg_v, [c16], e16)
                return carry2
            return lax.fori_loop(0, CG // L, sub, carry)
        lax.fori_loop(0, nchunk, chunk_body, 0)
        pltpu.sync_copy(deg_v, degp_hbm.at[pl.ds(w * npad, npad)])

    return deg_kernel


def _make_dinv_kernel(npad):
    npt = npad // NW

    @functools.partial(
        pl.kernel, mesh=_mesh(), compiler_params=_SC_PARAMS,
        out_type=jax.ShapeDtypeStruct((npad,), jnp.float32),
        scratch_types=[
            pltpu.VMEM((NW, npt), jnp.float32),
            pltpu.VMEM((npt,), jnp.float32),
        ])
    def dinv_kernel(degp_hbm, dinv_hbm, degs_v, dinv_v):
        w = lax.axis_index("s") * NC + lax.axis_index("c")
        for r in range(NW):
            pltpu.sync_copy(degp_hbm.at[pl.ds(r * npad + w * npt, npt)],
                            degs_v.at[r])

        def body(j, carry):
            acc = jnp.zeros((L,), jnp.float32)
            for r in range(NW):
                acc = acc + degs_v[r, pl.ds(j * L, L)]
            i32 = plsc.bitcast(acc, jnp.int32)
            i32 = jnp.int32(0x5F3759DF) - lax.shift_right_arithmetic(i32, 1)
            y = plsc.bitcast(i32, jnp.float32)
            for _ in range(3):
                y = y * (1.5 - 0.5 * acc * y * y)
            dinv_v[pl.ds(j * L, L)] = y
            return carry
        lax.fori_loop(0, npt // L, body, 0)
        pltpu.sync_copy(dinv_v, dinv_hbm.at[pl.ds(w * npt, npt)])

    return dinv_kernel


def _make_agg_kernel(nchunk, npad, npa):
    spt = npa // NS

    @functools.partial(
        pl.kernel, mesh=_mesh(), compiler_params=_SC_PARAMS,
        out_type=jax.ShapeDtypeStruct((NC, npa, D), jnp.float32),
        scratch_types=[
            pltpu.VMEM((CG,), jnp.int32),
            pltpu.VMEM((CG,), jnp.int32),
            pltpu.VMEM((CG,), jnp.float32),
            pltpu.VMEM((npa,), jnp.float32),
            pltpu.VMEM((CG,), jnp.float32),
            pltpu.VMEM((CG, D), jnp.float32),
            pltpu.VMEM_SHARED((npa, D), jnp.float32),
            pltpu.SemaphoreType.DMA,
        ])
    def agg_kernel(src_hbm, row_hbm, col_hbm, ew_hbm, dinv_hbm, out_hbm,
                   row_v, col_v, ew_v, dinv_v, norm_v, rows_v, accum, sem):
        cid = lax.axis_index("c")
        sid = lax.axis_index("s")
        w = sid * NC + cid
        pltpu.sync_copy(dinv_hbm.at[pl.ds(0, npa)], dinv_v)

        def zr(i, carry):
            for j in range(D // L):
                rows_v[i, pl.ds(j * L, L)] = jnp.zeros((L,), jnp.float32)
            return carry
        lax.fori_loop(0, CG, zr, 0)
        for b in range(spt // CG):
            pltpu.sync_copy(rows_v, accum.at[pl.ds(sid * spt + b * CG, CG)])
        plsc.subcore_barrier()

        def chunk(c, carry):
            pltpu.sync_copy(row_hbm.at[w, c], row_v)
            pltpu.sync_copy(col_hbm.at[w, c], col_v)
            pltpu.sync_copy(ew_hbm.at[w, c], ew_v)
            gat = pltpu.async_copy(src_hbm.at[row_v], rows_v, sem)
            def nb(j, carry2):
                r16 = row_v[pl.ds(j * L, L)]
                c16 = col_v[pl.ds(j * L, L)]
                e16 = ew_v[pl.ds(j * L, L)]
                dr = plsc.load_gather(dinv_v, [r16])
                dc = plsc.load_gather(dinv_v, [c16])
                norm_v[pl.ds(j * L, L)] = dr * e16 * dc
                return carry2
            lax.fori_loop(0, CG // L, nb, 0)
            gat.wait()

            def sc_e(e, carry2):
                nbv = plsc.load_gather(norm_v, [jnp.zeros((L,), jnp.int32) + e])
                for j in range(D // L):
                    rows_v[e, pl.ds(j * L, L)] = rows_v[e, pl.ds(j * L, L)] * nbv
                return carry2
            lax.fori_loop(0, CG, sc_e, 0)
            pltpu.sync_copy(rows_v, accum.at[col_v], add=True)
            return carry
        lax.fori_loop(0, nchunk, chunk, 0)
        plsc.subcore_barrier()
        pltpu.sync_copy(accum.at[pl.ds(sid * spt, spt)],
                        out_hbm.at[cid, pl.ds(sid * spt, spt)])

    return agg_kernel


def _relu_mm(p_ref, w_ref, b_ref, o_ref):
    a = p_ref[0] + p_ref[1]
    o_ref[...] = jnp.maximum(
        jnp.dot(a, w_ref[...], preferred_element_type=jnp.float32) + b_ref[...],
        0.0)


def _mm2(p_ref, w2_ref, b2_ref, w3_ref, b3_ref, m_ref, v_ref):
    a = p_ref[0] + p_ref[1]
    m_ref[...] = jnp.dot(a, w2_ref[...], preferred_element_type=jnp.float32) + b2_ref[...]
    v_ref[...] = jnp.dot(a, w3_ref[...], preferred_element_type=jnp.float32) + b3_ref[...]


def kernel(x, edge_index, edge_weight, W1, b1, W2, b2, W3, b3):
    n, d_in = x.shape
    e = edge_index.shape[1]
    npad = -(-n // 4096) * 4096
    npa = -(-n // (NS * CG)) * NS * CG
    etot = e + n
    nchunk = -(-etot // (NW * CG))
    epad = NW * CG * nchunk

    loop = jnp.arange(n, dtype=jnp.int32)
    row = jnp.concatenate([edge_index[0], loop])
    col = jnp.concatenate([edge_index[1], loop])
    ew = jnp.concatenate([edge_weight, jnp.ones((n,), jnp.float32)])
    pad = epad - etot
    row3 = jnp.pad(row, (0, pad)).reshape(NW, nchunk, CG)
    col3 = jnp.pad(col, (0, pad)).reshape(NW, nchunk, CG)
    ew3 = jnp.pad(ew, (0, pad)).reshape(NW, nchunk, CG)

    degp = _make_deg_kernel(nchunk, npad)(col3, ew3)
    dinv = _make_dinv_kernel(npad)(degp)

    agg = _make_agg_kernel(nchunk, npad, npa)
    p1 = agg(x, row3, col3, ew3, dinv)

    bn = 512
    h = pl.pallas_call(
        _relu_mm,
        grid=(npa // bn,),
        in_specs=[
            pl.BlockSpec((NC, bn, D), lambda i: (0, i, 0)),
            pl.BlockSpec((D, D), lambda i: (0, 0)),
            pl.BlockSpec((1, D), lambda i: (0, 0)),
        ],
        out_specs=pl.BlockSpec((bn, D), lambda i: (i, 0)),
        out_shape=jax.ShapeDtypeStruct((npa, D), jnp.float32),
    )(p1, W1, b1.reshape(1, D))

    p2 = agg(h, row3, col3, ew3, dinv)

    mean, var = pl.pallas_call(
        _mm2,
        grid=(npa // bn,),
        in_specs=[
            pl.BlockSpec((NC, bn, D), lambda i: (0, i, 0)),
            pl.BlockSpec((D, D), lambda i: (0, 0)),
            pl.BlockSpec((1, D), lambda i: (0, 0)),
            pl.BlockSpec((D, D), lambda i: (0, 0)),
            pl.BlockSpec((1, D), lambda i: (0, 0)),
        ],
        out_specs=[
            pl.BlockSpec((bn, D), lambda i: (i, 0)),
            pl.BlockSpec((bn, D), lambda i: (i, 0)),
        ],
        out_shape=[
            jax.ShapeDtypeStruct((npa, D), jnp.float32),
            jax.ShapeDtypeStruct((npa, D), jnp.float32),
        ],
    )(p2, W2, b2.reshape(1, D), W3, b3.reshape(1, D))

    return (mean[:n], var[:n])
